# baseline (device time: 136338 ns/iter reference)
import jax
import jax.numpy as jnp
from jax import lax
from jax.experimental import pallas as pl
from jax.experimental.pallas import tpu as pltpu

N_DEV = 16
M = 1536
N = 1536
M_PER = M // N_DEV


def kernel(A, B):

    def body(a_ref, b_ref, out_ref, p_ref, send_buf, recv_buf,
             send_sems, recv_sems):
        my = lax.axis_index("i")
        left = (my - 1) % N_DEV
        right = (my + 1) % N_DEV

        barrier_sem = pltpu.get_barrier_semaphore()
        for nbr in (left, right):
            pl.semaphore_signal(
                barrier_sem, inc=1,
                device_id=(nbr,), device_id_type=pl.DeviceIdType.MESH,
            )
        pl.semaphore_wait(barrier_sem, 2)

        p_ref[...] = jnp.dot(
            a_ref[...].astype(jnp.bfloat16),
            b_ref[...].astype(jnp.bfloat16),
            preferred_element_type=jnp.float32,
        )

        for t in range(N_DEV - 1):
            send_c = (my - 1 - t) % N_DEV
            local = p_ref[pl.ds(send_c * M_PER, M_PER), :]
            if t == 0:
                send_buf[t] = local
            else:
                send_buf[t] = recv_buf[t - 1] + local
            rdma = pltpu.make_async_remote_copy(
                src_ref=send_buf.at[t],
                dst_ref=recv_buf.at[t],
                send_sem=send_sems.at[t],
                recv_sem=recv_sems.at[t],
                device_id=(right,),
                device_id_type=pl.DeviceIdType.MESH,
            )
            rdma.start()
            rdma.wait()

        out_ref[...] = (
            recv_buf[N_DEV - 2]
            + p_ref[pl.ds(my * M_PER, M_PER), :]
        )

    n_steps = N_DEV - 1
    return pl.pallas_call(
        body,
        out_shape=jax.ShapeDtypeStruct((M_PER, N), jnp.float32),
        in_specs=[
            pl.BlockSpec(memory_space=pltpu.VMEM),
            pl.BlockSpec(memory_space=pltpu.VMEM),
        ],
        out_specs=pl.BlockSpec(memory_space=pltpu.VMEM),
        scratch_shapes=[
            pltpu.VMEM((M, N), jnp.float32),
            pltpu.VMEM((n_steps, M_PER, N), jnp.float32),
            pltpu.VMEM((n_steps, M_PER, N), jnp.float32),
            pltpu.SemaphoreType.DMA((n_steps,)),
            pltpu.SemaphoreType.DMA((n_steps,)),
        ],
        compiler_params=pltpu.CompilerParams(collective_id=0),
    )(A, B)


# device time: 63218 ns/iter; 2.1566x vs baseline; 2.1566x over previous
import jax
import jax.numpy as jnp
from jax import lax
from jax.experimental import pallas as pl
from jax.experimental.pallas import tpu as pltpu

N_DEV = 16
M = 1536
N = 1536
M_PER = M // N_DEV
FWD_STEPS = N_DEV // 2 - 1
BWD_STEPS = N_DEV // 2


def kernel(A, B):

    def body(a_ref, b_ref, out_ref, p_ref,
             fsend, frecv, bsend, brecv,
             fs_sems, fr_sems, bs_sems, br_sems):
        my = lax.axis_index("i")
        left = (my - 1) % N_DEV
        right = (my + 1) % N_DEV

        barrier_sem = pltpu.get_barrier_semaphore()
        for nbr in (left, right):
            pl.semaphore_signal(
                barrier_sem, inc=1,
                device_id=(nbr,), device_id_type=pl.DeviceIdType.MESH,
            )
        pl.semaphore_wait(barrier_sem, 2)

        p_ref[...] = jnp.dot(
            a_ref[...].astype(jnp.bfloat16),
            b_ref[...].astype(jnp.bfloat16),
            preferred_element_type=jnp.float32,
        )

        def pchunk(c):
            return p_ref[pl.ds(c * M_PER, M_PER), :]

        for t in range(BWD_STEPS):
            rdma_f = None
            if t < FWD_STEPS:
                d_fwd = (my + FWD_STEPS - t) % N_DEV
                acc = pchunk(d_fwd)
                if t > 0:
                    acc = acc + frecv[t - 1].astype(jnp.float32)
                fsend[t] = acc.astype(jnp.bfloat16)
                rdma_f = pltpu.make_async_remote_copy(
                    src_ref=fsend.at[t],
                    dst_ref=frecv.at[t],
                    send_sem=fs_sems.at[t],
                    recv_sem=fr_sems.at[t],
                    device_id=(right,),
                    device_id_type=pl.DeviceIdType.MESH,
                )
                rdma_f.start()
            d_bwd = (my - BWD_STEPS + t) % N_DEV
            acc = pchunk(d_bwd)
            if t > 0:
                acc = acc + brecv[t - 1].astype(jnp.float32)
            bsend[t] = acc.astype(jnp.bfloat16)
            rdma_b = pltpu.make_async_remote_copy(
                src_ref=bsend.at[t],
                dst_ref=brecv.at[t],
                send_sem=bs_sems.at[t],
                recv_sem=br_sems.at[t],
                device_id=(left,),
                device_id_type=pl.DeviceIdType.MESH,
            )
            rdma_b.start()
            if rdma_f is not None:
                rdma_f.wait()
            rdma_b.wait()

        out_ref[...] = (
            pchunk(my)
            + frecv[FWD_STEPS - 1].astype(jnp.float32)
            + brecv[BWD_STEPS - 1].astype(jnp.float32)
        )

    return pl.pallas_call(
        body,
        out_shape=jax.ShapeDtypeStruct((M_PER, N), jnp.float32),
        in_specs=[
            pl.BlockSpec(memory_space=pltpu.VMEM),
            pl.BlockSpec(memory_space=pltpu.VMEM),
        ],
        out_specs=pl.BlockSpec(memory_space=pltpu.VMEM),
        scratch_shapes=[
            pltpu.VMEM((M, N), jnp.float32),
            pltpu.VMEM((FWD_STEPS, M_PER, N), jnp.bfloat16),
            pltpu.VMEM((FWD_STEPS, M_PER, N), jnp.bfloat16),
            pltpu.VMEM((BWD_STEPS, M_PER, N), jnp.bfloat16),
            pltpu.VMEM((BWD_STEPS, M_PER, N), jnp.bfloat16),
            pltpu.SemaphoreType.DMA((FWD_STEPS,)),
            pltpu.SemaphoreType.DMA((FWD_STEPS,)),
            pltpu.SemaphoreType.DMA((BWD_STEPS,)),
            pltpu.SemaphoreType.DMA((BWD_STEPS,)),
        ],
        compiler_params=pltpu.CompilerParams(collective_id=0),
    )(A, B)
